# baseline (speedup 1.0000x reference)
import functools

import jax
import jax.numpy as jnp
from jax.experimental import pallas as pl
from jax.experimental.pallas import tpu as pltpu


def _body(x0_ref, h_ref, w0_ref, w1_ref, x0_out_ref, x1_out_ref,
          xcur_ref, acc_ref, *, nj, bj, e):
    l = pl.program_id(0)
    j = pl.program_id(1)

    @pl.when(jnp.logical_and(l == 0, j == 0))
    def _():
        xcur_ref[...] = x0_ref[...].astype(jnp.bfloat16)

    stripe = h_ref[...].astype(jnp.bfloat16)
    if e % bj != 0:
        col = jax.lax.broadcasted_iota(jnp.int32, (1, bj), 1) + j * bj
        stripe = jnp.where(col < e, stripe, jnp.bfloat16(0.0))

    x1t = jax.lax.dot_general(
        stripe, xcur_ref[...], (((0,), (0,)), ((), ())),
        preferred_element_type=jnp.float32)
    x1_out_ref[...] = x1t

    w = jnp.where(l == 0, w0_ref[...], w1_ref[...]).astype(jnp.bfloat16)
    y = jnp.dot(x1t.astype(jnp.bfloat16), w,
                preferred_element_type=jnp.float32)

    contrib = jax.lax.dot_general(
        stripe, y.astype(jnp.bfloat16), (((1,), (0,)), ((), ())),
        preferred_element_type=jnp.float32)

    @pl.when(j == 0)
    def _():
        acc_ref[...] = contrib

    @pl.when(j > 0)
    def _():
        acc_ref[...] += contrib

    @pl.when(jnp.logical_and(l == 0, j == nj - 1))
    def _():
        xcur_ref[...] = acc_ref[...].astype(jnp.bfloat16)

    @pl.when(jnp.logical_and(l == 1, j == nj - 1))
    def _():
        x0_out_ref[...] = acc_ref[...]


def kernel(x_0, incidence_1, W0, W1):
    n, c = x_0.shape
    e = incidence_1.shape[1]
    bj = 384
    nj = -(-e // bj)

    grid = (2, nj)
    out_shape = (
        jax.ShapeDtypeStruct((n, c), jnp.float32),
        jax.ShapeDtypeStruct((e, c), jnp.float32),
    )
    x0_out, x1_out = pl.pallas_call(
        functools.partial(_body, nj=nj, bj=bj, e=e),
        grid=grid,
        in_specs=[
            pl.BlockSpec((n, c), lambda l, j: (0, 0)),
            pl.BlockSpec((n, bj), lambda l, j: (0, j)),
            pl.BlockSpec((c, c), lambda l, j: (0, 0)),
            pl.BlockSpec((c, c), lambda l, j: (0, 0)),
        ],
        out_specs=[
            pl.BlockSpec((n, c), lambda l, j: (0, 0)),
            pl.BlockSpec((bj, c), lambda l, j: (j, 0)),
        ],
        out_shape=out_shape,
        scratch_shapes=[
            pltpu.VMEM((n, c), jnp.bfloat16),
            pltpu.VMEM((n, c), jnp.float32),
        ],
        compiler_params=pltpu.CompilerParams(
            dimension_semantics=("arbitrary", "arbitrary")),
    )(x_0, incidence_1, W0, W1)
    return x0_out, x1_out


# speedup vs baseline: 1.6882x; 1.6882x over previous
import functools

import jax
import jax.numpy as jnp
from jax.experimental import pallas as pl
from jax.experimental.pallas import tpu as pltpu


def _body(x0t_ref, h_ref, w0_ref, w1_ref, x0_out_ref, x1_out_ref,
          xcurt_ref, acc_ref, *, nj, bj, e):
    l = pl.program_id(0)
    j = pl.program_id(1)

    @pl.when(jnp.logical_and(l == 0, j == 0))
    def _():
        xcurt_ref[...] = x0t_ref[...]

    stripe = h_ref[...].astype(jnp.bfloat16)

    x1tt = jnp.dot(xcurt_ref[...], stripe,
                   preferred_element_type=jnp.float32)
    x1t = x1tt.T
    if e % bj != 0:
        row = jax.lax.broadcasted_iota(jnp.int32, (bj, 1), 0) + j * bj
        x1t = jnp.where(row < e, x1t, 0.0)
    x1_out_ref[...] = x1t

    w = jnp.where(l == 0, w0_ref[...], w1_ref[...]).astype(jnp.bfloat16)
    y = jnp.dot(x1t.astype(jnp.bfloat16), w,
                preferred_element_type=jnp.float32)

    contrib = jnp.dot(stripe, y.astype(jnp.bfloat16),
                      preferred_element_type=jnp.float32)

    @pl.when(j == 0)
    def _():
        acc_ref[...] = contrib

    @pl.when(j > 0)
    def _():
        acc_ref[...] += contrib

    @pl.when(jnp.logical_and(l == 0, j == nj - 1))
    def _():
        xcurt_ref[...] = acc_ref[...].T.astype(jnp.bfloat16)

    @pl.when(jnp.logical_and(l == 1, j == nj - 1))
    def _():
        x0_out_ref[...] = acc_ref[...]


def kernel(x_0, incidence_1, W0, W1):
    n, c = x_0.shape
    e = incidence_1.shape[1]
    bj = 384
    nj = -(-e // bj)

    x0t = x_0.T.astype(jnp.bfloat16)

    grid = (2, nj)
    out_shape = (
        jax.ShapeDtypeStruct((n, c), jnp.float32),
        jax.ShapeDtypeStruct((e, c), jnp.float32),
    )
    x0_out, x1_out = pl.pallas_call(
        functools.partial(_body, nj=nj, bj=bj, e=e),
        grid=grid,
        in_specs=[
            pl.BlockSpec((c, n), lambda l, j: (0, 0)),
            pl.BlockSpec((n, bj), lambda l, j: (0, j)),
            pl.BlockSpec((c, c), lambda l, j: (0, 0)),
            pl.BlockSpec((c, c), lambda l, j: (0, 0)),
        ],
        out_specs=[
            pl.BlockSpec((n, c), lambda l, j: (0, 0)),
            pl.BlockSpec((bj, c), lambda l, j: (j, 0)),
        ],
        out_shape=out_shape,
        scratch_shapes=[
            pltpu.VMEM((c, n), jnp.bfloat16),
            pltpu.VMEM((n, c), jnp.float32),
        ],
        compiler_params=pltpu.CompilerParams(
            dimension_semantics=("arbitrary", "arbitrary")),
    )(x0t, incidence_1, W0, W1)
    return x0_out, x1_out


# speedup vs baseline: 1.9722x; 1.1682x over previous
import functools

import jax
import jax.numpy as jnp
from jax.experimental import pallas as pl
from jax.experimental.pallas import tpu as pltpu


def _body(x0t_ref, h_ref, w0_ref, w1_ref, x0_out_ref, x1_out_ref,
          *, nj, bj, e):
    l = pl.program_id(0)
    j = pl.program_id(1)

    stripe = h_ref[...].astype(jnp.bfloat16)

    x1tt = jnp.dot(x0t_ref[...], stripe,
                   preferred_element_type=jnp.float32)
    x1t = x1tt.T
    if e % bj != 0:
        row = jax.lax.broadcasted_iota(jnp.int32, (bj, 1), 0) + j * bj
        x1t = jnp.where(row < e, x1t, 0.0)
    x1_out_ref[...] = x1t

    w = jnp.where(l == 0, w0_ref[...], w1_ref[...]).astype(jnp.bfloat16)
    y = jnp.dot(x1t.astype(jnp.bfloat16), w,
                preferred_element_type=jnp.float32)

    contrib = jnp.dot(stripe, y.astype(jnp.bfloat16),
                      preferred_element_type=jnp.float32)

    @pl.when(j == 0)
    def _():
        x0_out_ref[...] = contrib

    @pl.when(j > 0)
    def _():
        x0_out_ref[...] += contrib

    @pl.when(jnp.logical_and(l == 0, j == nj - 1))
    def _():
        x0t_ref[...] = x0_out_ref[...].T.astype(jnp.bfloat16)


def kernel(x_0, incidence_1, W0, W1):
    n, c = x_0.shape
    e = incidence_1.shape[1]
    bj = 512
    nj = -(-e // bj)

    x0t = x_0.T.astype(jnp.bfloat16)

    grid = (2, nj)
    out_shape = (
        jax.ShapeDtypeStruct((n, c), jnp.float32),
        jax.ShapeDtypeStruct((e, c), jnp.float32),
    )
    x0_out, x1_out = pl.pallas_call(
        functools.partial(_body, nj=nj, bj=bj, e=e),
        grid=grid,
        in_specs=[
            pl.BlockSpec((c, n), lambda l, j: (0, 0)),
            pl.BlockSpec((n, bj), lambda l, j: (0, j)),
            pl.BlockSpec((c, c), lambda l, j: (0, 0)),
            pl.BlockSpec((c, c), lambda l, j: (0, 0)),
        ],
        out_specs=[
            pl.BlockSpec((n, c), lambda l, j: (0, 0)),
            pl.BlockSpec((bj, c), lambda l, j: (j, 0)),
        ],
        out_shape=out_shape,
        compiler_params=pltpu.CompilerParams(
            dimension_semantics=("arbitrary", "arbitrary")),
    )(x0t, incidence_1, W0, W1)
    return x0_out, x1_out
